# Initial kernel scaffold; baseline (speedup 1.0000x reference)
import jax
import jax.numpy as jnp
from jax.experimental import pallas as pl


def kernel(emb, input_ids):
    raise NotImplementedError("write your pallas kernel here")


# speedup vs baseline: 7.7782x; 7.7782x over previous
import functools

import jax
import jax.numpy as jnp
from jax import lax
from jax.experimental import pallas as pl
from jax.experimental.pallas import tpu as pltpu
from jax.experimental.pallas import tpu_sc as plsc

D = 16
LANES = 128
MIN_NORM = 1e-15
BALL_EPS = 4e-3
ATANH_CLIP = 1.0 - 1e-7


def _group_norm_sq(x):
    ii = lax.broadcasted_iota(jnp.int32, (LANES, LANES), 0) // D
    jj = lax.broadcasted_iota(jnp.int32, (LANES, LANES), 1) // D
    m = (ii == jj).astype(jnp.float32)
    return lax.dot(x * x, m, precision=lax.Precision.HIGHEST)


def _logmap_body(x_ref, o_ref):
    x = x_ref[...]
    n = jnp.maximum(jnp.sqrt(_group_norm_sq(x)), MIN_NORM)
    a = jnp.minimum(n, ATANH_CLIP)
    f = (0.5 * jnp.log((1.0 + a) / (1.0 - a))) / n
    o_ref[...] = x * f


def _finalize_body(x_ref, o_ref, *, inv_l):
    mean = x_ref[...] * inv_l
    n = jnp.maximum(jnp.sqrt(_group_norm_sq(mean)), MIN_NORM)
    t = jnp.tanh(n)
    f = jnp.minimum(t, 1.0 - BALL_EPS) / n
    o_ref[...] = mean * f


def _logmap_table(emb):
    v, d = emb.shape
    flat = emb.reshape(v * d // LANES, LANES)
    rows = flat.shape[0]
    blk = 1000
    grid = rows // blk
    out = pl.pallas_call(
        _logmap_body,
        grid=(grid,),
        in_specs=[pl.BlockSpec((blk, LANES), lambda i: (i, 0))],
        out_specs=pl.BlockSpec((blk, LANES), lambda i: (i, 0)),
        out_shape=jax.ShapeDtypeStruct((rows, LANES), jnp.float32),
    )(flat)
    return out.reshape(v, d)


def _finalize(sums, seq_len):
    b, d = sums.shape
    flat = sums.reshape(b * d // LANES, LANES)
    rows = flat.shape[0]
    blk = min(256, rows)
    out = pl.pallas_call(
        functools.partial(_finalize_body, inv_l=1.0 / seq_len),
        grid=(rows // blk,),
        in_specs=[pl.BlockSpec((blk, LANES), lambda i: (i, 0))],
        out_specs=pl.BlockSpec((blk, LANES), lambda i: (i, 0)),
        out_shape=jax.ShapeDtypeStruct((rows, LANES), jnp.float32),
    )(flat)
    return out.reshape(b, d)


def _sc_gather_sum(tang, ids2d, batch, seq_len):
    n_cores, n_sub = 2, 16
    nw = n_cores * n_sub
    ex_w = batch // nw
    ech = 16
    nstep = ex_w // ech
    rows_c = ech * seq_len
    ksub = rows_c // 128
    ids3d = ids2d.reshape(nw * nstep, ksub, 128)
    mesh = plsc.VectorSubcoreMesh(core_axis_name="c", subcore_axis_name="s")

    @functools.partial(
        pl.kernel,
        mesh=mesh,
        compiler_params=pltpu.CompilerParams(use_tc_tiling_on_sc=False),
        out_type=jax.ShapeDtypeStruct((batch, D), jnp.float32),
        scratch_types=[
            pltpu.VMEM((2, ksub, 128), jnp.int32),
            pltpu.VMEM((2, rows_c, D), jnp.float32),
            pltpu.VMEM((ex_w, D), jnp.float32),
            pltpu.SemaphoreType.DMA,
            pltpu.SemaphoreType.DMA,
        ],
    )
    def k(tang_hbm, ids_hbm, out_hbm, idx_v, rows_v, out_v, sem0, sem1):
        sems = (sem0, sem1)
        wid = lax.axis_index("c") * n_sub + lax.axis_index("s")

        def start_load(s, b):
            chunk = wid * nstep + s
            pltpu.sync_copy(ids_hbm.at[chunk], idx_v.at[b])
            for j in range(ksub):
                pltpu.async_copy(
                    tang_hbm.at[idx_v.at[b, j]],
                    rows_v.at[b, pl.ds(j * 128, 128)],
                    sems[b],
                )

        def wait_rows(b):
            pltpu.make_async_copy(
                tang_hbm.at[pl.ds(0, rows_c)], rows_v.at[b], sems[b]
            ).wait()

        def sum_example(b, base):
            zero = jnp.zeros((D,), jnp.float32)

            def tbody(i, accs):
                a0, a1, a2, a3 = accs
                o = base + i * 8
                a0 = a0 + rows_v[b, o]
                a1 = a1 + rows_v[b, o + 1]
                a2 = a2 + rows_v[b, o + 2]
                a3 = a3 + rows_v[b, o + 3]
                a0 = a0 + rows_v[b, o + 4]
                a1 = a1 + rows_v[b, o + 5]
                a2 = a2 + rows_v[b, o + 6]
                a3 = a3 + rows_v[b, o + 7]
                return a0, a1, a2, a3

            a0, a1, a2, a3 = lax.fori_loop(
                0, seq_len // 8, tbody, (zero, zero, zero, zero)
            )
            return (a0 + a1) + (a2 + a3)

        start_load(0, 0)
        start_load(1, 1)

        def step(s0, carry):
            for b in range(2):
                s = s0 * 2 + b
                wait_rows(b)
                for e in range(ech):
                    out_v[s * ech + e] = sum_example(b, e * seq_len)

                @pl.when(s + 2 < nstep)
                def _():
                    start_load(s + 2, b)
            return carry

        lax.fori_loop(0, nstep // 2, step, 0)
        pltpu.sync_copy(out_v, out_hbm.at[pl.ds(wid * ex_w, ex_w)])

    return k(tang, ids3d)


def kernel(emb, input_ids):
    batch, seq_len = input_ids.shape
    tang = _logmap_table(emb)
    ids2d = input_ids.astype(jnp.int32).reshape(batch * seq_len // 128, 128)
    sums = _sc_gather_sum(tang, ids2d, batch, seq_len)
    return _finalize(sums, seq_len)
